# Initial kernel scaffold; baseline (speedup 1.0000x reference)
import jax
import jax.numpy as jnp
from jax.experimental import pallas as pl


def kernel(bias_table, seq_len):
    raise NotImplementedError("write your pallas kernel here")


# speedup vs baseline: 42.6001x; 42.6001x over previous
import functools

import jax
import jax.numpy as jnp
from jax import lax
from jax.experimental import pallas as pl
from jax.experimental.pallas import tpu as pltpu
from jax.experimental.pallas import tpu_sc as plsc

NUM_HEADS = 16
SEQ = 2048
TBL = 1023
TBL_FLAT = TBL * NUM_HEADS
EXT_PITCH = 4352
NUM_SHIFTS = 8
LANES = 16
ROWS_PER_WORKER = SEQ // 2
FIRE = 16
CHUNKS = ROWS_PER_WORKER // FIRE


def _body(table_hbm, out_hbm, tbl_v, ext_v, sem):
    head = lax.axis_index("s")
    half = lax.axis_index("c")
    row_base = half * ROWS_PER_WORKER

    pltpu.sync_copy(table_hbm, tbl_v.at[pl.ds(0, TBL_FLAT)])

    lane = lax.iota(jnp.int32, LANES)

    def build(it, _):
        base = it * LANES
        pos = base + lane
        for b in range(NUM_SHIFTS):
            r = jnp.clip(pos + (b - 1536), 0, TBL - 1)
            vals = plsc.load_gather(tbl_v, [r * NUM_HEADS + head])
            ext_v[pl.ds(b * EXT_PITCH + base, LANES)] = vals
        return 0

    lax.fori_loop(0, EXT_PITCH // LANES, build, 0)

    def chunk(c, _):
        copies = []
        for j in range(FIRE):
            i = row_base + c * FIRE + j
            q = (SEQ - 1) - i
            b = lax.rem(q, NUM_SHIFTS)
            src_off = pl.multiple_of(b * EXT_PITCH + (q - b), 8)
            dst_off = pl.multiple_of((head * SEQ + i) * SEQ, SEQ)
            copies.append(pltpu.async_copy(
                ext_v.at[pl.ds(src_off, SEQ)],
                out_hbm.at[pl.ds(dst_off, SEQ)],
                sem))
        for cp in copies:
            cp.wait()
        return 0

    lax.fori_loop(0, CHUNKS, chunk, 0)


@jax.jit
def _materialize(table_flat):
    f = functools.partial(
        pl.kernel,
        out_type=jax.ShapeDtypeStruct((NUM_HEADS * SEQ * SEQ,), jnp.float32),
        mesh=plsc.VectorSubcoreMesh(core_axis_name="c", subcore_axis_name="s"),
        scratch_types=[
            pltpu.VMEM((16384,), jnp.float32),
            pltpu.VMEM((NUM_SHIFTS * EXT_PITCH,), jnp.float32),
            pltpu.SemaphoreType.DMA,
        ],
        compiler_params=pltpu.CompilerParams(needs_layout_passes=False),
    )(_body)
    return f(table_flat)


def kernel(bias_table, seq_len):
    del seq_len
    out_flat = _materialize(bias_table.reshape(-1))
    return out_flat.reshape(NUM_HEADS, SEQ, SEQ)
